# baseline (speedup 1.0000x reference)
import jax
import jax.numpy as jnp
from jax.experimental import pallas as pl
from jax.experimental.pallas import tpu as pltpu

_LB = 256


def _rows_body(mask_ref, pred_ref, z_ref, pi_ref, zi_ref, po_ref, zo_ref):
    b = pl.program_id(0)
    m = mask_ref[b] != 0
    pi = jnp.broadcast_to(pi_ref[0], po_ref.shape)
    zi = jnp.broadcast_to(zi_ref[0], zo_ref.shape)
    po_ref[...] = jnp.where(m, pi, pred_ref[...])
    zo_ref[...] = jnp.where(m, zi, z_ref[...])


def _steps_body(mask_ref, steps_ref, out_ref):
    out_ref[...] = jnp.where(mask_ref[...] != 0, jnp.int32(0), steps_ref[...])


def kernel(prediction_y, reasoning_Z, steps, mask, pred_init, Z_init):
    B, L, D = prediction_y.shape
    mask_i = mask.astype(jnp.int32)
    grid = (B, L // _LB)
    blk = pl.BlockSpec((1, _LB, D), lambda b, j: (b, j, 0))
    init_blk = pl.BlockSpec((1, 1, D), lambda b, j: (0, 0, 0))
    pred_out, Z_out = pl.pallas_call(
        _rows_body,
        grid=grid,
        in_specs=[
            pl.BlockSpec(memory_space=pltpu.SMEM),
            blk,
            blk,
            init_blk,
            init_blk,
        ],
        out_specs=[blk, blk],
        out_shape=[jax.ShapeDtypeStruct((B, L, D), jnp.float32)] * 2,
    )(mask_i, prediction_y, reasoning_Z, pred_init, Z_init)

    steps_out = pl.pallas_call(
        _steps_body,
        out_shape=jax.ShapeDtypeStruct((1, B), jnp.int32),
    )(mask_i.reshape(1, B), steps.reshape(1, B))
    return (pred_out, Z_out, steps_out.reshape(B))
